# baseline (speedup 1.0000x reference)
import jax
import jax.numpy as jnp
from jax import lax
from jax.experimental import pallas as pl
from jax.experimental.pallas import tpu as pltpu
from jax.experimental.pallas import tpu_sc as plsc

V = 1000000
D = 64
B = 16384
C = 20
NEG = 3
CN = C + NEG
NC = 2
NS = 16
NW = NC * NS
BPW = B // NW
CHUNK = 16
NSTEPS = BPW // CHUNK
IDXN = CHUNK * CN + 16


def _gather_body(tidx_hbm, cidx_hbm, tbl_hbm, out_hbm,
                 tidx_v, cidx_v, cpk0_v, cpk1_v,
                 ctx_rows0, ctx_rows1,
                 out_v0, out_v1, sem_c0, sem_c1):
    cid = lax.axis_index("c")
    sid = lax.axis_index("s")
    wid = sid * NC + cid
    base = wid * BPW

    pltpu.sync_copy(tidx_hbm.at[pl.ds(base, BPW)],
                    tidx_v.at[pl.ds(0, BPW)])
    tidx_v[pl.ds(BPW, 16)] = jnp.zeros((16,), jnp.int32)
    pltpu.sync_copy(cidx_hbm.at[pl.ds(base * CN, BPW * CN)],
                    cidx_v.at[pl.ds(0, BPW * CN)])

    ctx_rows = (ctx_rows0, ctx_rows1)
    outs = (out_v0, out_v1)
    sems_c = (sem_c0, sem_c1)
    cpks = (cpk0_v, cpk1_v)

    def issue(s, b):
        s0 = s * (CHUNK * CN)
        for w in range(CHUNK * CN // 16):
            cpks[b][pl.ds(w * 16, 16)] = \
                lax.shift_right_logical(cidx_v[pl.ds(s0 + w * 16, 16)], 1)
        if CHUNK * CN % 16:
            tl = CHUNK * CN - 16
            cpks[b][pl.ds(tl, 16)] = \
                lax.shift_right_logical(cidx_v[pl.ds(s0 + tl, 16)], 1)
        half = jnp.full((16,), V // 2, jnp.int32)
        cpks[b][pl.ds(CHUNK * CN, 16)] = half + lax.shift_right_logical(
            tidx_v[pl.ds(s * CHUNK, 16)], 1)
        pltpu.async_copy(tbl_hbm.at[cpks[b]], ctx_rows[b], sems_c[b])

    def wait_bufs(b):
        pltpu.make_async_copy(tbl_hbm.at[cpks[b]],
                              ctx_rows[b], sems_c[b]).wait()

    def step_b(s, b):
        ib = base + s * CHUNK

        @pl.when(s + 1 < NSTEPS)
        def _():
            issue(s + 1, 1 - b)

        wait_bufs(b)
        crv = ctx_rows[b]
        ov = outs[b]

        def item(i, carry2):
            ib23 = i * CN
            gi23 = s * (CHUNK * CN) + ib23
            t = tidx_v[pl.ds(s * CHUNK + i, 16)][0]
            pt = (t & 1) * 64
            tr = CHUNK * CN + i
            t0 = crv[tr, pl.ds(pt, 16)]
            t1 = crv[tr, pl.ds(pt + 16, 16)]
            t2 = crv[tr, pl.ds(pt + 32, 16)]
            t3 = crv[tr, pl.ds(pt + 48, 16)]
            cs0 = jnp.zeros((16,), jnp.float32)
            cs1 = jnp.zeros((16,), jnp.float32)
            cs2 = jnp.zeros((16,), jnp.float32)
            cs3 = jnp.zeros((16,), jnp.float32)
            for j in range(C):
                pc = (cidx_v[pl.ds(gi23 + j, 16)][0] & 1) * 64
                cs0 = cs0 + crv[ib23 + j, pl.ds(pc, 16)]
                cs1 = cs1 + crv[ib23 + j, pl.ds(pc + 16, 16)]
                cs2 = cs2 + crv[ib23 + j, pl.ds(pc + 32, 16)]
                cs3 = cs3 + crv[ib23 + j, pl.ds(pc + 48, 16)]
            ns0 = jnp.zeros((16,), jnp.float32)
            ns1 = jnp.zeros((16,), jnp.float32)
            ns2 = jnp.zeros((16,), jnp.float32)
            ns3 = jnp.zeros((16,), jnp.float32)
            for j in range(C, CN):
                pn = (cidx_v[pl.ds(gi23 + j, 16)][0] & 1) * 64
                ns0 = ns0 + crv[ib23 + j, pl.ds(pn, 16)]
                ns1 = ns1 + crv[ib23 + j, pl.ds(pn + 16, 16)]
                ns2 = ns2 + crv[ib23 + j, pl.ds(pn + 32, 16)]
                ns3 = ns3 + crv[ib23 + j, pl.ds(pn + 48, 16)]
            pacc = cs0 * t0 + cs1 * t1 + cs2 * t2 + cs3 * t3
            nacc = ns0 * t0 + ns1 * t1 + ns2 * t2 + ns3 * t3
            ov[i, pl.ds(0, 16)] = pacc
            ov[i, pl.ds(16, 16)] = nacc
            return carry2

        lax.fori_loop(0, CHUNK, item, 0, unroll=False)
        pltpu.sync_copy(ov, out_hbm.at[pl.ds(ib, CHUNK)])

    issue(0, 0)

    def step(s, carry):
        b = lax.rem(s, 2)

        @pl.when(b == 0)
        def _():
            step_b(s, 0)

        @pl.when(b == 1)
        def _():
            step_b(s, 1)

        return carry

    lax.fori_loop(0, NSTEPS, step, 0, unroll=False)


def _tc_body(part_ref, out_ref):
    x = part_ref[...]
    p = jnp.sum(x[:, :16], axis=1) * (1.0 / C)
    n = -jnp.sum(x[:, 16:], axis=1)

    def logsig(v):
        return jnp.minimum(v, 0.0) - jnp.log1p(jnp.exp(-jnp.abs(v)))

    total = jnp.sum(logsig(p) + logsig(n))
    out_ref[0, 0] = -total * (1.0 / B)


def kernel(targets, contexts, negsamples, context_emb, target_emb):
    tidx = targets.astype(jnp.int32)
    cidx = jnp.concatenate(
        [contexts.astype(jnp.int32), negsamples.astype(jnp.int32)],
        axis=1).reshape(B * CN)
    tbl = jnp.concatenate([context_emb.reshape(V // 2, 2 * D),
                           target_emb.reshape(V // 2, 2 * D)], axis=0)

    mesh = plsc.VectorSubcoreMesh(core_axis_name="c", subcore_axis_name="s",
                                  num_cores=NC, num_subcores=NS)
    gather = pl.kernel(
        _gather_body,
        out_type=jax.ShapeDtypeStruct((B, 32), jnp.float32),
        mesh=mesh,
        scratch_types=[
            pltpu.VMEM((BPW + 16,), jnp.int32),
            pltpu.VMEM((BPW * CN + 16,), jnp.int32),
            pltpu.VMEM((IDXN,), jnp.int32),
            pltpu.VMEM((IDXN,), jnp.int32),
            pltpu.VMEM((IDXN, 2 * D), jnp.float32),
            pltpu.VMEM((IDXN, 2 * D), jnp.float32),
            pltpu.VMEM((CHUNK, 32), jnp.float32),
            pltpu.VMEM((CHUNK, 32), jnp.float32),
            pltpu.SemaphoreType.DMA,
            pltpu.SemaphoreType.DMA,
        ],
    )
    part = gather(tidx, cidx, tbl)

    loss = pl.pallas_call(
        _tc_body,
        out_shape=jax.ShapeDtypeStruct((1, 1), jnp.float32),
        in_specs=[pl.BlockSpec(memory_space=pltpu.VMEM)],
        out_specs=pl.BlockSpec(memory_space=pltpu.SMEM),
    )(part)
    return loss


# speedup vs baseline: 1.6045x; 1.6045x over previous
import jax
import jax.numpy as jnp
from jax import lax
from jax.experimental import pallas as pl
from jax.experimental.pallas import tpu as pltpu
from jax.experimental.pallas import tpu_sc as plsc

V = 1000000
D = 64
B = 16384
C = 20
NEG = 3
CN = C + NEG
NC = 2
NS = 16
NW = NC * NS
BPW = B // NW
CHUNK = 16
NSTEPS = BPW // CHUNK
IDXN = CHUNK * CN + 16


def _gather_body(cidx_hbm, ctx_tbl_hbm, tgt_sel_hbm, out_hbm,
                 cidx_v, cpk0_v, cpk1_v,
                 ctx_rows0, ctx_rows1, tgt_rows0, tgt_rows1,
                 out_v0, out_v1, sem_c0, sem_c1, sem_t0, sem_t1):
    cid = lax.axis_index("c")
    sid = lax.axis_index("s")
    wid = sid * NC + cid
    base = wid * BPW

    pltpu.sync_copy(cidx_hbm.at[pl.ds(base * CN, BPW * CN)],
                    cidx_v.at[pl.ds(0, BPW * CN)])

    ctx_rows = (ctx_rows0, ctx_rows1)
    tgt_rows = (tgt_rows0, tgt_rows1)
    outs = (out_v0, out_v1)
    sems_c = (sem_c0, sem_c1)
    sems_t = (sem_t0, sem_t1)
    cpks = (cpk0_v, cpk1_v)

    def issue(s, b):
        s0 = s * (CHUNK * CN)
        for w in range(CHUNK * CN // 16):
            cpks[b][pl.ds(w * 16, 16)] = \
                lax.shift_right_logical(cidx_v[pl.ds(s0 + w * 16, 16)], 1)
        if CHUNK * CN % 16:
            tl = CHUNK * CN - 16
            cpks[b][pl.ds(tl, 16)] = \
                lax.shift_right_logical(cidx_v[pl.ds(s0 + tl, 16)], 1)
        pltpu.async_copy(ctx_tbl_hbm.at[cpks[b]], ctx_rows[b], sems_c[b])
        pltpu.async_copy(tgt_sel_hbm.at[pl.ds(base + s * CHUNK, CHUNK), :],
                         tgt_rows[b], sems_t[b])

    def wait_bufs(b):
        pltpu.make_async_copy(ctx_tbl_hbm.at[cpks[b]],
                              ctx_rows[b], sems_c[b]).wait()
        pltpu.make_async_copy(tgt_sel_hbm.at[pl.ds(0, CHUNK), :],
                              tgt_rows[b], sems_t[b]).wait()

    def step_b(s, b):
        ib = base + s * CHUNK

        @pl.when(s + 1 < NSTEPS)
        def _():
            issue(s + 1, 1 - b)

        wait_bufs(b)
        crv = ctx_rows[b]
        trv = tgt_rows[b]
        ov = outs[b]

        def item(i, carry2):
            ib23 = i * CN
            gi23 = s * (CHUNK * CN) + ib23
            t0 = trv[i, pl.ds(0, 16)]
            t1 = trv[i, pl.ds(16, 16)]
            t2 = trv[i, pl.ds(32, 16)]
            t3 = trv[i, pl.ds(48, 16)]
            cs0 = jnp.zeros((16,), jnp.float32)
            cs1 = jnp.zeros((16,), jnp.float32)
            cs2 = jnp.zeros((16,), jnp.float32)
            cs3 = jnp.zeros((16,), jnp.float32)
            for j in range(C):
                pc = (cidx_v[pl.ds(gi23 + j, 16)][0] & 1) * 64
                cs0 = cs0 + crv[ib23 + j, pl.ds(pc, 16)]
                cs1 = cs1 + crv[ib23 + j, pl.ds(pc + 16, 16)]
                cs2 = cs2 + crv[ib23 + j, pl.ds(pc + 32, 16)]
                cs3 = cs3 + crv[ib23 + j, pl.ds(pc + 48, 16)]
            ns0 = jnp.zeros((16,), jnp.float32)
            ns1 = jnp.zeros((16,), jnp.float32)
            ns2 = jnp.zeros((16,), jnp.float32)
            ns3 = jnp.zeros((16,), jnp.float32)
            for j in range(C, CN):
                pn = (cidx_v[pl.ds(gi23 + j, 16)][0] & 1) * 64
                ns0 = ns0 + crv[ib23 + j, pl.ds(pn, 16)]
                ns1 = ns1 + crv[ib23 + j, pl.ds(pn + 16, 16)]
                ns2 = ns2 + crv[ib23 + j, pl.ds(pn + 32, 16)]
                ns3 = ns3 + crv[ib23 + j, pl.ds(pn + 48, 16)]
            pacc = cs0 * t0 + cs1 * t1 + cs2 * t2 + cs3 * t3
            nacc = ns0 * t0 + ns1 * t1 + ns2 * t2 + ns3 * t3
            ov[i, pl.ds(0, 16)] = pacc
            ov[i, pl.ds(16, 16)] = nacc
            return carry2

        lax.fori_loop(0, CHUNK, item, 0, unroll=False)
        pltpu.sync_copy(ov, out_hbm.at[pl.ds(ib, CHUNK)])

    issue(0, 0)

    def step(s, carry):
        b = lax.rem(s, 2)

        @pl.when(b == 0)
        def _():
            step_b(s, 0)

        @pl.when(b == 1)
        def _():
            step_b(s, 1)

        return carry

    lax.fori_loop(0, NSTEPS, step, 0, unroll=False)


def _tc_body(part_ref, out_ref):
    x = part_ref[...]
    p = jnp.sum(x[:, :16], axis=1) * (1.0 / C)
    n = -jnp.sum(x[:, 16:], axis=1)

    def logsig(v):
        return jnp.minimum(v, 0.0) - jnp.log1p(jnp.exp(-jnp.abs(v)))

    total = jnp.sum(logsig(p) + logsig(n))
    out_ref[0, 0] = -total * (1.0 / B)


def kernel(targets, contexts, negsamples, context_emb, target_emb):
    cidx = jnp.concatenate(
        [contexts.astype(jnp.int32), negsamples.astype(jnp.int32)],
        axis=1).reshape(B * CN)
    ctx_tbl = context_emb.reshape(V // 2, 2 * D)
    tgt_sel = jnp.take(target_emb, targets, axis=0)

    mesh = plsc.VectorSubcoreMesh(core_axis_name="c", subcore_axis_name="s",
                                  num_cores=NC, num_subcores=NS)
    gather = pl.kernel(
        _gather_body,
        out_type=jax.ShapeDtypeStruct((B, 32), jnp.float32),
        mesh=mesh,
        scratch_types=[
            pltpu.VMEM((BPW * CN + 16,), jnp.int32),
            pltpu.VMEM((CHUNK * CN,), jnp.int32),
            pltpu.VMEM((CHUNK * CN,), jnp.int32),
            pltpu.VMEM((CHUNK * CN, 2 * D), jnp.float32),
            pltpu.VMEM((CHUNK * CN, 2 * D), jnp.float32),
            pltpu.VMEM((CHUNK, D), jnp.float32),
            pltpu.VMEM((CHUNK, D), jnp.float32),
            pltpu.VMEM((CHUNK, 32), jnp.float32),
            pltpu.VMEM((CHUNK, 32), jnp.float32),
            pltpu.SemaphoreType.DMA,
            pltpu.SemaphoreType.DMA,
            pltpu.SemaphoreType.DMA,
            pltpu.SemaphoreType.DMA,
        ],
    )
    part = gather(cidx, ctx_tbl, tgt_sel)

    loss = pl.pallas_call(
        _tc_body,
        out_shape=jax.ShapeDtypeStruct((1, 1), jnp.float32),
        in_specs=[pl.BlockSpec(memory_space=pltpu.VMEM)],
        out_specs=pl.BlockSpec(memory_space=pltpu.SMEM),
    )(part)
    return loss


# speedup vs baseline: 1.6430x; 1.0240x over previous
import jax
import jax.numpy as jnp
from jax import lax
from jax.experimental import pallas as pl
from jax.experimental.pallas import tpu as pltpu
from jax.experimental.pallas import tpu_sc as plsc

V = 1000000
D = 64
B = 16384
C = 20
NEG = 3
CN = C + NEG
NC = 2
NS = 16
NW = NC * NS
BPW = B // NW
CHUNK = 16
NSTEPS = BPW // CHUNK


def _gather_body(cidx_hbm, ctx_hbm, tgt_sel_hbm, out_hbm,
                 cidx_v,
                 ctx_rows0, ctx_rows1, tgt_rows0, tgt_rows1,
                 out_v0, out_v1, sem_c0, sem_c1, sem_t0, sem_t1):
    cid = lax.axis_index("c")
    sid = lax.axis_index("s")
    wid = sid * NC + cid
    base = wid * BPW

    pltpu.sync_copy(cidx_hbm.at[pl.ds(base * CN, BPW * CN)], cidx_v)

    ctx_rows = (ctx_rows0, ctx_rows1)
    tgt_rows = (tgt_rows0, tgt_rows1)
    outs = (out_v0, out_v1)
    sems_c = (sem_c0, sem_c1)
    sems_t = (sem_t0, sem_t1)

    def issue(s, b):
        pltpu.async_copy(
            ctx_hbm.at[cidx_v.at[pl.ds(s * (CHUNK * CN), CHUNK * CN)]],
            ctx_rows[b], sems_c[b])
        pltpu.async_copy(tgt_sel_hbm.at[pl.ds(base + s * CHUNK, CHUNK), :],
                         tgt_rows[b], sems_t[b])

    def wait_bufs(b):
        pltpu.make_async_copy(
            ctx_hbm.at[cidx_v.at[pl.ds(0, CHUNK * CN)]],
            ctx_rows[b], sems_c[b]).wait()
        pltpu.make_async_copy(tgt_sel_hbm.at[pl.ds(0, CHUNK), :],
                              tgt_rows[b], sems_t[b]).wait()

    def step_b(s, b):
        ib = base + s * CHUNK

        @pl.when(s + 1 < NSTEPS)
        def _():
            issue(s + 1, 1 - b)

        wait_bufs(b)
        crv = ctx_rows[b]
        trv = tgt_rows[b]
        ov = outs[b]

        def item(i, carry2):
            ib23 = i * CN
            t0 = trv[i, pl.ds(0, 16)]
            t1 = trv[i, pl.ds(16, 16)]
            t2 = trv[i, pl.ds(32, 16)]
            t3 = trv[i, pl.ds(48, 16)]
            cs0 = jnp.zeros((16,), jnp.float32)
            cs1 = jnp.zeros((16,), jnp.float32)
            cs2 = jnp.zeros((16,), jnp.float32)
            cs3 = jnp.zeros((16,), jnp.float32)
            for j in range(C):
                cs0 = cs0 + crv[ib23 + j, pl.ds(0, 16)]
                cs1 = cs1 + crv[ib23 + j, pl.ds(16, 16)]
                cs2 = cs2 + crv[ib23 + j, pl.ds(32, 16)]
                cs3 = cs3 + crv[ib23 + j, pl.ds(48, 16)]
            ns0 = jnp.zeros((16,), jnp.float32)
            ns1 = jnp.zeros((16,), jnp.float32)
            ns2 = jnp.zeros((16,), jnp.float32)
            ns3 = jnp.zeros((16,), jnp.float32)
            for j in range(C, CN):
                ns0 = ns0 + crv[ib23 + j, pl.ds(0, 16)]
                ns1 = ns1 + crv[ib23 + j, pl.ds(16, 16)]
                ns2 = ns2 + crv[ib23 + j, pl.ds(32, 16)]
                ns3 = ns3 + crv[ib23 + j, pl.ds(48, 16)]
            pacc = cs0 * t0 + cs1 * t1 + cs2 * t2 + cs3 * t3
            nacc = ns0 * t0 + ns1 * t1 + ns2 * t2 + ns3 * t3
            ov[i, pl.ds(0, 16)] = pacc
            ov[i, pl.ds(16, 16)] = nacc
            return carry2

        lax.fori_loop(0, CHUNK, item, 0, unroll=False)
        pltpu.sync_copy(ov, out_hbm.at[pl.ds(ib, CHUNK)])

    issue(0, 0)

    def step(s, carry):
        b = lax.rem(s, 2)

        @pl.when(b == 0)
        def _():
            step_b(s, 0)

        @pl.when(b == 1)
        def _():
            step_b(s, 1)

        return carry

    lax.fori_loop(0, NSTEPS, step, 0, unroll=False)


def _tc_body(part_ref, out_ref):
    x = part_ref[...]
    p = jnp.sum(x[:, :16], axis=1) * (1.0 / C)
    n = -jnp.sum(x[:, 16:], axis=1)

    def logsig(v):
        return jnp.minimum(v, 0.0) - jnp.log1p(jnp.exp(-jnp.abs(v)))

    total = jnp.sum(logsig(p) + logsig(n))
    out_ref[0, 0] = -total * (1.0 / B)


def kernel(targets, contexts, negsamples, context_emb, target_emb):
    cidx = jnp.concatenate(
        [contexts.astype(jnp.int32), negsamples.astype(jnp.int32)],
        axis=1).reshape(B * CN)
    tgt_sel = jnp.take(target_emb, targets, axis=0)

    mesh = plsc.VectorSubcoreMesh(core_axis_name="c", subcore_axis_name="s",
                                  num_cores=NC, num_subcores=NS)
    gather = pl.kernel(
        _gather_body,
        out_type=jax.ShapeDtypeStruct((B, 32), jnp.float32),
        mesh=mesh,
        compiler_params=pltpu.CompilerParams(use_tc_tiling_on_sc=False),
        scratch_types=[
            pltpu.VMEM((BPW * CN,), jnp.int32),
            pltpu.VMEM((CHUNK * CN, D), jnp.float32),
            pltpu.VMEM((CHUNK * CN, D), jnp.float32),
            pltpu.VMEM((CHUNK, D), jnp.float32),
            pltpu.VMEM((CHUNK, D), jnp.float32),
            pltpu.VMEM((CHUNK, 32), jnp.float32),
            pltpu.VMEM((CHUNK, 32), jnp.float32),
            pltpu.SemaphoreType.DMA,
            pltpu.SemaphoreType.DMA,
            pltpu.SemaphoreType.DMA,
            pltpu.SemaphoreType.DMA,
        ],
    )
    part = gather(cidx, context_emb, tgt_sel)

    loss = pl.pallas_call(
        _tc_body,
        out_shape=jax.ShapeDtypeStruct((1, 1), jnp.float32),
        in_specs=[pl.BlockSpec(memory_space=pltpu.VMEM)],
        out_specs=pl.BlockSpec(memory_space=pltpu.SMEM),
    )(part)
    return loss
